# baseline (speedup 1.0000x reference)
import functools

import numpy as np
import jax
import jax.numpy as jnp
from jax import lax
from jax.experimental import pallas as pl
from jax.experimental.pallas import tpu as pltpu
from jax.experimental.pallas import tpu_sc as plsc

_NOISE_SCALE = 0.1
_NOISE_MIN = 0.0001
_NOISE_MAX = 0.02
_STEPS = 1000
_TAB_PAD = 1024


def _diffusion_coef_tables():
    start = _NOISE_SCALE * _NOISE_MIN
    end = _NOISE_SCALE * _NOISE_MAX
    variance = np.linspace(start, end, _STEPS, dtype=np.float64)
    alpha_bar = 1.0 - variance
    betas = [1.0 - alpha_bar[0]]
    for i in range(1, _STEPS):
        betas.append(min(1.0 - alpha_bar[i] / alpha_bar[i - 1], 0.999))
    betas = np.array(betas, dtype=np.float64)
    betas[0] = 0.0001
    alphas = 1.0 - betas
    acp = np.cumprod(alphas, axis=0)
    a = np.sqrt(acp)
    b = np.sqrt(1.0 - acp)
    pad = _TAB_PAD - _STEPS
    a = np.pad(a, (0, pad)).astype(np.float32)
    b = np.pad(b, (0, pad)).astype(np.float32)
    return a, b


_TAB_A, _TAB_B = _diffusion_coef_tables()

_B, _D = 16384, 128
_NC, _NS, _L = 2, 16, 16
_NW = _NC * _NS
_RPW = _B // _NW
_CHUNK = 128
_NCHUNK = _RPW // _CHUNK


def _sc_body(a_hbm, b_hbm, ts_hbm, x_hbm, n_hbm, o_hbm,
             ts_v, ca_v, cb_v, xb, nb, ob, sg, sx, sn, so):
    wid = lax.axis_index("s") * _NC + lax.axis_index("c")
    base = wid * _RPW

    n_idx_rows = _RPW // 128
    for j in range(n_idx_rows):
        pltpu.sync_copy(ts_hbm.at[pl.ds(base + j * 128, 128)], ts_v.at[j])
    for j in range(n_idx_rows):
        pltpu.async_copy(a_hbm.at[ts_v.at[j]], ca_v.at[pl.ds(j * 128, 128)], sg)
        pltpu.async_copy(b_hbm.at[ts_v.at[j]], cb_v.at[pl.ds(j * 128, 128)], sg)
    for j in range(n_idx_rows):
        pltpu.make_async_copy(a_hbm.at[ts_v.at[j]], ca_v.at[pl.ds(j * 128, 128)], sg).wait()
        pltpu.make_async_copy(b_hbm.at[ts_v.at[j]], cb_v.at[pl.ds(j * 128, 128)], sg).wait()

    def start_in(c, s):
        r0 = base + c * _CHUNK
        pltpu.async_copy(x_hbm.at[pl.ds(r0, _CHUNK)], xb.at[s], sx)
        pltpu.async_copy(n_hbm.at[pl.ds(r0, _CHUNK)], nb.at[s], sn)

    def wait_in(c, s):
        r0 = base + c * _CHUNK
        pltpu.make_async_copy(x_hbm.at[pl.ds(r0, _CHUNK)], xb.at[s], sx).wait()
        pltpu.make_async_copy(n_hbm.at[pl.ds(r0, _CHUNK)], nb.at[s], sn).wait()

    def start_out(c, s):
        r0 = base + c * _CHUNK
        pltpu.async_copy(ob.at[s], o_hbm.at[pl.ds(r0, _CHUNK)], so)

    def wait_out(c, s):
        r0 = base + c * _CHUNK
        pltpu.make_async_copy(ob.at[s], o_hbm.at[pl.ds(r0, _CHUNK)], so).wait()

    start_in(0, 0)

    def ring_step(c0, carry):
        for s in range(2):
            c = c0 * 2 + s

            @pl.when(c + 1 < _NCHUNK)
            def _():
                start_in(c + 1, 1 - s)

            wait_in(c, s)

            @pl.when(c >= 2)
            def _():
                wait_out(c - 2, s)

            @plsc.parallel_loop(0, _CHUNK // _L, 1, unroll=2)
            def row_group(g):
                gsl = pl.ds(c * _CHUNK + g * _L, _L)
                ca_vec = ca_v[gsl]
                cb_vec = cb_v[gsl]
                for j in range(_L):
                    r = g * _L + j
                    ca = jnp.full((_L,), ca_vec[j])
                    cb = jnp.full((_L,), cb_vec[j])
                    for k in range(_D // _L):
                        sl = pl.ds(k * _L, _L)
                        ob[s, r, sl] = ca * xb[s, r, sl] + cb * nb[s, r, sl]
            start_out(c, s)
        return carry

    lax.fori_loop(0, _NCHUNK // 2, ring_step, 0)
    wait_out(_NCHUNK - 2, 0)
    wait_out(_NCHUNK - 1, 1)


@functools.partial(jax.jit, static_argnames=("interpret",))
def kernel(x_start, ts, noise, interpret=False):
    mesh = plsc.VectorSubcoreMesh(
        core_axis_name="c", subcore_axis_name="s",
        num_cores=_NC, num_subcores=_NS)
    ker = pl.kernel(
        _sc_body,
        out_type=jax.ShapeDtypeStruct((_B, _D), jnp.float32),
        mesh=mesh,
        scratch_types=[
            pltpu.VMEM((_RPW // 128, 128), jnp.int32),
            pltpu.VMEM((_RPW,), jnp.float32),
            pltpu.VMEM((_RPW,), jnp.float32),
            pltpu.VMEM((2, _CHUNK, _D), jnp.float32),
            pltpu.VMEM((2, _CHUNK, _D), jnp.float32),
            pltpu.VMEM((2, _CHUNK, _D), jnp.float32),
            pltpu.SemaphoreType.DMA,
            pltpu.SemaphoreType.DMA,
            pltpu.SemaphoreType.DMA,
            pltpu.SemaphoreType.DMA,
        ],
        interpret=interpret,
    )
    return ker(jnp.asarray(_TAB_A), jnp.asarray(_TAB_B),
               ts.astype(jnp.int32), x_start, noise)


# speedup vs baseline: 1.0233x; 1.0233x over previous
import functools

import numpy as np
import jax
import jax.numpy as jnp
from jax import lax
from jax.experimental import pallas as pl
from jax.experimental.pallas import tpu as pltpu
from jax.experimental.pallas import tpu_sc as plsc

_NOISE_SCALE = 0.1
_NOISE_MIN = 0.0001
_NOISE_MAX = 0.02
_STEPS = 1000
_TAB_PAD = 1024


def _diffusion_coef_tables():
    start = _NOISE_SCALE * _NOISE_MIN
    end = _NOISE_SCALE * _NOISE_MAX
    variance = np.linspace(start, end, _STEPS, dtype=np.float64)
    alpha_bar = 1.0 - variance
    betas = [1.0 - alpha_bar[0]]
    for i in range(1, _STEPS):
        betas.append(min(1.0 - alpha_bar[i] / alpha_bar[i - 1], 0.999))
    betas = np.array(betas, dtype=np.float64)
    betas[0] = 0.0001
    alphas = 1.0 - betas
    acp = np.cumprod(alphas, axis=0)
    a = np.sqrt(acp)
    b = np.sqrt(1.0 - acp)
    pad = _TAB_PAD - _STEPS
    a = np.pad(a, (0, pad)).astype(np.float32)
    b = np.pad(b, (0, pad)).astype(np.float32)
    return a, b


_TAB_A, _TAB_B = _diffusion_coef_tables()

_B, _D = 16384, 128
_NC, _NS, _L = 2, 16, 16
_NW = _NC * _NS
_RPW = _B // _NW
_CHUNK = 64
_NCHUNK = _RPW // _CHUNK


def _sc_body(a_hbm, b_hbm, ts_hbm, x_hbm, n_hbm, o_hbm,
             ts_v, ca_v, cb_v, xb, nb, ob, sg, sx, sn, so):
    wid = lax.axis_index("s") * _NC + lax.axis_index("c")
    base = wid * _RPW

    def start_in(c, s):
        r0 = base + c * _CHUNK
        pltpu.async_copy(x_hbm.at[pl.ds(r0, _CHUNK)], xb.at[s], sx)
        pltpu.async_copy(n_hbm.at[pl.ds(r0, _CHUNK)], nb.at[s], sn)

    def wait_in(c, s):
        r0 = base + c * _CHUNK
        pltpu.make_async_copy(x_hbm.at[pl.ds(r0, _CHUNK)], xb.at[s], sx).wait()
        pltpu.make_async_copy(n_hbm.at[pl.ds(r0, _CHUNK)], nb.at[s], sn).wait()

    def start_out(c, s):
        r0 = base + c * _CHUNK
        pltpu.async_copy(ob.at[s], o_hbm.at[pl.ds(r0, _CHUNK)], so)

    def wait_out(c, s):
        r0 = base + c * _CHUNK
        pltpu.make_async_copy(ob.at[s], o_hbm.at[pl.ds(r0, _CHUNK)], so).wait()

    start_in(0, 0)
    start_in(1, 1)
    n_idx_rows = _RPW // 128
    for j in range(n_idx_rows):
        pltpu.async_copy(ts_hbm.at[pl.ds(base + j * 128, 128)], ts_v.at[j], sg)
    for j in range(n_idx_rows):
        pltpu.make_async_copy(ts_hbm.at[pl.ds(base + j * 128, 128)], ts_v.at[j], sg).wait()
    for j in range(n_idx_rows):
        pltpu.async_copy(a_hbm.at[ts_v.at[j]], ca_v.at[pl.ds(j * 128, 128)], sg)
        pltpu.async_copy(b_hbm.at[ts_v.at[j]], cb_v.at[pl.ds(j * 128, 128)], sg)
    for j in range(n_idx_rows):
        pltpu.make_async_copy(a_hbm.at[ts_v.at[j]], ca_v.at[pl.ds(j * 128, 128)], sg).wait()
        pltpu.make_async_copy(b_hbm.at[ts_v.at[j]], cb_v.at[pl.ds(j * 128, 128)], sg).wait()

    def ring_step(c0, carry):
        for s in range(2):
            c = c0 * 2 + s

            wait_in(c, s)

            @pl.when(c >= 2)
            def _():
                wait_out(c - 2, s)

            @plsc.parallel_loop(0, _CHUNK // _L, 1, unroll=2)
            def row_group(g):
                gsl = pl.ds(c * _CHUNK + g * _L, _L)
                ca_vec = ca_v[gsl]
                cb_vec = cb_v[gsl]
                for j in range(_L):
                    r = g * _L + j
                    ca = jnp.full((_L,), ca_vec[j])
                    cb = jnp.full((_L,), cb_vec[j])
                    for k in range(_D // _L):
                        sl = pl.ds(k * _L, _L)
                        ob[s, r, sl] = ca * xb[s, r, sl] + cb * nb[s, r, sl]
            start_out(c, s)

            @pl.when(c + 2 < _NCHUNK)
            def _():
                start_in(c + 2, s)
        return carry

    lax.fori_loop(0, _NCHUNK // 2, ring_step, 0)
    wait_out(_NCHUNK - 2, 0)
    wait_out(_NCHUNK - 1, 1)


@functools.partial(jax.jit, static_argnames=("interpret",))
def kernel(x_start, ts, noise, interpret=False):
    mesh = plsc.VectorSubcoreMesh(
        core_axis_name="c", subcore_axis_name="s",
        num_cores=_NC, num_subcores=_NS)
    ker = pl.kernel(
        _sc_body,
        out_type=jax.ShapeDtypeStruct((_B, _D), jnp.float32),
        mesh=mesh,
        scratch_types=[
            pltpu.VMEM((_RPW // 128, 128), jnp.int32),
            pltpu.VMEM((_RPW,), jnp.float32),
            pltpu.VMEM((_RPW,), jnp.float32),
            pltpu.VMEM((2, _CHUNK, _D), jnp.float32),
            pltpu.VMEM((2, _CHUNK, _D), jnp.float32),
            pltpu.VMEM((2, _CHUNK, _D), jnp.float32),
            pltpu.SemaphoreType.DMA,
            pltpu.SemaphoreType.DMA,
            pltpu.SemaphoreType.DMA,
            pltpu.SemaphoreType.DMA,
        ],
        interpret=interpret,
    )
    return ker(jnp.asarray(_TAB_A), jnp.asarray(_TAB_B),
               ts.astype(jnp.int32), x_start, noise)


# speedup vs baseline: 1.1352x; 1.1094x over previous
import functools

import numpy as np
import jax
import jax.numpy as jnp
from jax import lax
from jax.experimental import pallas as pl
from jax.experimental.pallas import tpu as pltpu
from jax.experimental.pallas import tpu_sc as plsc

_NOISE_SCALE = 0.1
_NOISE_MIN = 0.0001
_NOISE_MAX = 0.02
_STEPS = 1000
_TAB_PAD = 1024


def _diffusion_coef_tables():
    start = _NOISE_SCALE * _NOISE_MIN
    end = _NOISE_SCALE * _NOISE_MAX
    variance = np.linspace(start, end, _STEPS, dtype=np.float64)
    alpha_bar = 1.0 - variance
    betas = [1.0 - alpha_bar[0]]
    for i in range(1, _STEPS):
        betas.append(min(1.0 - alpha_bar[i] / alpha_bar[i - 1], 0.999))
    betas = np.array(betas, dtype=np.float64)
    betas[0] = 0.0001
    alphas = 1.0 - betas
    acp = np.cumprod(alphas, axis=0)
    a = np.sqrt(acp)
    b = np.sqrt(1.0 - acp)
    pad = _TAB_PAD - _STEPS
    a = np.pad(a, (0, pad)).astype(np.float32)
    b = np.pad(b, (0, pad)).astype(np.float32)
    return a, b


_TAB_A, _TAB_B = _diffusion_coef_tables()

_B, _D = 16384, 128
_NC, _NS, _L = 2, 16, 16
_NW = _NC * _NS
_RPW = _B // _NW
_CHUNK = 64
_NCHUNK = _RPW // _CHUNK


def _sc_body(a_hbm, b_hbm, ts_hbm, x_hbm, n_hbm, o_hbm,
             ts_v, ca_v, cb_v, xb, nb, ob, sg, sx, sn, so):
    wid = lax.axis_index("s") * _NC + lax.axis_index("c")
    base = wid * _RPW

    def start_in(c, s):
        r0 = base + c * _CHUNK
        pltpu.async_copy(x_hbm.at[pl.ds(r0, _CHUNK)], xb.at[s], sx)
        pltpu.async_copy(n_hbm.at[pl.ds(r0, _CHUNK)], nb.at[s], sn)

    def wait_in(c, s):
        r0 = base + c * _CHUNK
        pltpu.make_async_copy(x_hbm.at[pl.ds(r0, _CHUNK)], xb.at[s], sx).wait()
        pltpu.make_async_copy(n_hbm.at[pl.ds(r0, _CHUNK)], nb.at[s], sn).wait()

    def start_out(c, s):
        r0 = base + c * _CHUNK
        pltpu.async_copy(ob.at[s], o_hbm.at[pl.ds(r0, _CHUNK)], so)

    def wait_out(c, s):
        r0 = base + c * _CHUNK
        pltpu.make_async_copy(ob.at[s], o_hbm.at[pl.ds(r0, _CHUNK)], so).wait()

    start_in(0, 0)
    start_in(1, 1)
    n_idx_rows = _RPW // 128
    for j in range(n_idx_rows):
        pltpu.async_copy(ts_hbm.at[pl.ds(base + j * 128, 128)], ts_v.at[j], sg)
    for j in range(n_idx_rows):
        pltpu.make_async_copy(ts_hbm.at[pl.ds(base + j * 128, 128)], ts_v.at[j], sg).wait()
    for j in range(n_idx_rows):
        pltpu.async_copy(a_hbm.at[ts_v.at[j]], ca_v.at[pl.ds(j * 128, 128)], sg)
        pltpu.async_copy(b_hbm.at[ts_v.at[j]], cb_v.at[pl.ds(j * 128, 128)], sg)
    for j in range(n_idx_rows):
        pltpu.make_async_copy(a_hbm.at[ts_v.at[j]], ca_v.at[pl.ds(j * 128, 128)], sg).wait()
        pltpu.make_async_copy(b_hbm.at[ts_v.at[j]], cb_v.at[pl.ds(j * 128, 128)], sg).wait()

    def ring_step(c0, carry):
        for s in range(2):
            c = c0 * 2 + s

            wait_in(c, s)

            @pl.when(c >= 2)
            def _():
                wait_out(c - 2, s)

            def row_group(g, rcarry):
                gsl = pl.ds(c * _CHUNK + g * _L, _L)
                ca_vec = ca_v[gsl]
                cb_vec = cb_v[gsl]
                for j in range(_L):
                    r = g * _L + j
                    ca = jnp.full((_L,), ca_vec[j])
                    cb = jnp.full((_L,), cb_vec[j])
                    for k in range(_D // _L):
                        sl = pl.ds(k * _L, _L)
                        ob[s, r, sl] = ca * xb[s, r, sl] + cb * nb[s, r, sl]
                return rcarry

            lax.fori_loop(0, _CHUNK // _L, row_group, 0)
            start_out(c, s)

            @pl.when(c + 2 < _NCHUNK)
            def _():
                start_in(c + 2, s)
        return carry

    lax.fori_loop(0, _NCHUNK // 2, ring_step, 0)
    wait_out(_NCHUNK - 2, 0)
    wait_out(_NCHUNK - 1, 1)


@functools.partial(jax.jit, static_argnames=("interpret",))
def kernel(x_start, ts, noise, interpret=False):
    mesh = plsc.VectorSubcoreMesh(
        core_axis_name="c", subcore_axis_name="s",
        num_cores=_NC, num_subcores=_NS)
    ker = pl.kernel(
        _sc_body,
        out_type=jax.ShapeDtypeStruct((_B, _D), jnp.float32),
        mesh=mesh,
        scratch_types=[
            pltpu.VMEM((_RPW // 128, 128), jnp.int32),
            pltpu.VMEM((_RPW,), jnp.float32),
            pltpu.VMEM((_RPW,), jnp.float32),
            pltpu.VMEM((2, _CHUNK, _D), jnp.float32),
            pltpu.VMEM((2, _CHUNK, _D), jnp.float32),
            pltpu.VMEM((2, _CHUNK, _D), jnp.float32),
            pltpu.SemaphoreType.DMA,
            pltpu.SemaphoreType.DMA,
            pltpu.SemaphoreType.DMA,
            pltpu.SemaphoreType.DMA,
        ],
        interpret=interpret,
    )
    return ker(jnp.asarray(_TAB_A), jnp.asarray(_TAB_B),
               ts.astype(jnp.int32), x_start, noise)


# speedup vs baseline: 1.5578x; 1.3722x over previous
import functools

import numpy as np
import jax
import jax.numpy as jnp
from jax import lax
from jax.experimental import pallas as pl
from jax.experimental.pallas import tpu as pltpu
from jax.experimental.pallas import tpu_sc as plsc

_NOISE_SCALE = 0.1
_NOISE_MIN = 0.0001
_NOISE_MAX = 0.02
_STEPS = 1000
_TAB_PAD = 1024


def _diffusion_coef_tables():
    start = _NOISE_SCALE * _NOISE_MIN
    end = _NOISE_SCALE * _NOISE_MAX
    variance = np.linspace(start, end, _STEPS, dtype=np.float64)
    alpha_bar = 1.0 - variance
    betas = [1.0 - alpha_bar[0]]
    for i in range(1, _STEPS):
        betas.append(min(1.0 - alpha_bar[i] / alpha_bar[i - 1], 0.999))
    betas = np.array(betas, dtype=np.float64)
    betas[0] = 0.0001
    alphas = 1.0 - betas
    acp = np.cumprod(alphas, axis=0)
    a = np.sqrt(acp)
    b = np.sqrt(1.0 - acp)
    pad = _TAB_PAD - _STEPS
    a = np.pad(a, (0, pad)).astype(np.float32)
    b = np.pad(b, (0, pad)).astype(np.float32)
    return np.concatenate(
        [np.repeat(a[:, None], 16, axis=1),
         np.repeat(b[:, None], 16, axis=1)], axis=1)


_TAB_AB = _diffusion_coef_tables()

_B, _D = 16384, 128
_NC, _NS, _L = 2, 16, 16
_NW = _NC * _NS
_RPW = _B // _NW
_CHUNK = 64
_NCHUNK = _RPW // _CHUNK


def _sc_body(t_hbm, ts_hbm, x_hbm, n_hbm, o_hbm,
             ts_v, ct_v, xb, nb, ob, sg, sx, sn, so):
    wid = lax.axis_index("s") * _NC + lax.axis_index("c")
    base = wid * _RPW

    def start_in(c, s):
        r0 = base + c * _CHUNK
        pltpu.async_copy(x_hbm.at[pl.ds(r0, _CHUNK)], xb.at[s], sx)
        pltpu.async_copy(n_hbm.at[pl.ds(r0, _CHUNK)], nb.at[s], sn)

    def wait_in(c, s):
        r0 = base + c * _CHUNK
        pltpu.make_async_copy(x_hbm.at[pl.ds(r0, _CHUNK)], xb.at[s], sx).wait()
        pltpu.make_async_copy(n_hbm.at[pl.ds(r0, _CHUNK)], nb.at[s], sn).wait()

    def start_out(c, s):
        r0 = base + c * _CHUNK
        pltpu.async_copy(ob.at[s], o_hbm.at[pl.ds(r0, _CHUNK)], so)

    def wait_out(c, s):
        r0 = base + c * _CHUNK
        pltpu.make_async_copy(ob.at[s], o_hbm.at[pl.ds(r0, _CHUNK)], so).wait()

    start_in(0, 0)
    start_in(1, 1)
    n_idx_rows = _RPW // 128
    for j in range(n_idx_rows):
        pltpu.async_copy(ts_hbm.at[pl.ds(base + j * 128, 128)], ts_v.at[j], sg)
    for j in range(n_idx_rows):
        pltpu.make_async_copy(ts_hbm.at[pl.ds(base + j * 128, 128)], ts_v.at[j], sg).wait()
    for j in range(n_idx_rows):
        pltpu.async_copy(t_hbm.at[ts_v.at[j]], ct_v.at[pl.ds(j * 128, 128)], sg)
    for j in range(n_idx_rows):
        pltpu.make_async_copy(t_hbm.at[ts_v.at[j]], ct_v.at[pl.ds(j * 128, 128)], sg).wait()

    def ring_step(c0, carry):
        for s in range(2):
            c = c0 * 2 + s

            wait_in(c, s)

            @pl.when(c >= 2)
            def _():
                wait_out(c - 2, s)

            def row(r, rcarry):
                tr = c * _CHUNK + r
                ca = ct_v[tr, pl.ds(0, _L)]
                cb = ct_v[tr, pl.ds(_L, _L)]
                for k in range(_D // _L):
                    sl = pl.ds(k * _L, _L)
                    ob[s, r, sl] = ca * xb[s, r, sl] + cb * nb[s, r, sl]
                return rcarry

            lax.fori_loop(0, _CHUNK, row, 0)
            start_out(c, s)

            @pl.when(c + 2 < _NCHUNK)
            def _():
                start_in(c + 2, s)
        return carry

    lax.fori_loop(0, _NCHUNK // 2, ring_step, 0)
    wait_out(_NCHUNK - 2, 0)
    wait_out(_NCHUNK - 1, 1)


@functools.partial(jax.jit, static_argnames=("interpret",))
def kernel(x_start, ts, noise, interpret=False):
    mesh = plsc.VectorSubcoreMesh(
        core_axis_name="c", subcore_axis_name="s",
        num_cores=_NC, num_subcores=_NS)
    ker = pl.kernel(
        _sc_body,
        out_type=jax.ShapeDtypeStruct((_B, _D), jnp.float32),
        mesh=mesh,
        scratch_types=[
            pltpu.VMEM((_RPW // 128, 128), jnp.int32),
            pltpu.VMEM((_RPW, 2 * _L), jnp.float32),
            pltpu.VMEM((2, _CHUNK, _D), jnp.float32),
            pltpu.VMEM((2, _CHUNK, _D), jnp.float32),
            pltpu.VMEM((2, _CHUNK, _D), jnp.float32),
            pltpu.SemaphoreType.DMA,
            pltpu.SemaphoreType.DMA,
            pltpu.SemaphoreType.DMA,
            pltpu.SemaphoreType.DMA,
        ],
        compiler_params=pltpu.CompilerParams(use_tc_tiling_on_sc=False),
        interpret=interpret,
    )
    return ker(jnp.asarray(_TAB_AB),
               ts.astype(jnp.int32), x_start, noise)
